# baseline (speedup 1.0000x reference)
import jax
import jax.numpy as jnp
from jax import lax
from jax.experimental import pallas as pl
from jax.experimental.pallas import tpu as pltpu

CHUNK_ROWS = 512
NBUF = 3


def _make_kernel(rows, cols, inv_n):
    n_chunks = rows // CHUNK_ROWS
    n_rounds = n_chunks // NBUF

    def body(x_hbm, t_hbm, o_ref, xb, tb, sx, st):
        def start(c, slot):
            r = c * CHUNK_ROWS
            pltpu.make_async_copy(
                x_hbm.at[pl.ds(r, CHUNK_ROWS)], xb.at[slot], sx.at[slot]
            ).start()
            pltpu.make_async_copy(
                t_hbm.at[pl.ds(r, CHUNK_ROWS)], tb.at[slot], st.at[slot]
            ).start()

        def wait(slot):
            pltpu.make_async_copy(
                x_hbm.at[pl.ds(0, CHUNK_ROWS)], xb.at[slot], sx.at[slot]
            ).wait()
            pltpu.make_async_copy(
                t_hbm.at[pl.ds(0, CHUNK_ROWS)], tb.at[slot], st.at[slot]
            ).wait()

        def compute(slot, acc):
            t = tb[slot]
            x = xb[slot]
            safe_t = jnp.where(t > 0, t, 1.0)
            kl = t * jnp.log(safe_t) - t * x
            return acc + jnp.sum(kl)

        for slot in range(NBUF):
            start(slot, slot)

        def round_body(p, acc):
            c0 = p * NBUF
            for slot in range(NBUF):
                wait(slot)
                acc = compute(slot, acc)
                nxt = c0 + slot + NBUF

                @pl.when(nxt < n_chunks)
                def _(nxt=nxt, slot=slot):
                    start(nxt, slot)

            return acc

        acc = lax.fori_loop(0, n_rounds, round_body, jnp.float32(0.0))
        tail = n_chunks - n_rounds * NBUF
        for k in range(tail):
            slot = (n_rounds * NBUF + k) % NBUF
            wait(slot)
            acc = compute(slot, acc)
        o_ref[0, 0] = acc * inv_n

    return body


def kernel(inputs, targets):
    rows, cols = inputs.shape

    out = pl.pallas_call(
        _make_kernel(rows, cols, 1.0 / (rows * cols)),
        in_specs=[
            pl.BlockSpec(memory_space=pl.ANY),
            pl.BlockSpec(memory_space=pl.ANY),
        ],
        out_specs=pl.BlockSpec(memory_space=pltpu.SMEM),
        out_shape=jax.ShapeDtypeStruct((1, 1), jnp.float32),
        scratch_shapes=[
            pltpu.VMEM((NBUF, CHUNK_ROWS, 4096), jnp.float32),
            pltpu.VMEM((NBUF, CHUNK_ROWS, 4096), jnp.float32),
            pltpu.SemaphoreType.DMA((NBUF,)),
            pltpu.SemaphoreType.DMA((NBUF,)),
        ],
    )(inputs, targets)
    return out.reshape(())
